# baseline (speedup 1.0000x reference)
import jax
import jax.numpy as jnp
from jax.experimental import pallas as pl
from jax.experimental.pallas import tpu as pltpu
from jax.experimental.pallas import tpu_sc as plsc

W = 168
D = 512
B = 16384
NW = 32
BPW = B // NW
CH = 64
LANES = 16


def kernel(t, cycleQueue):
    t32 = t.astype(jnp.int32)
    mesh = plsc.VectorSubcoreMesh(core_axis_name="core", subcore_axis_name="subcore")

    @pl.kernel(
        out_type=jax.ShapeDtypeStruct((B, D), jnp.float32),
        mesh=mesh,
        scratch_types=[
            pltpu.VMEM((BPW,), jnp.int32),
            pltpu.VMEM((2, CH, D), jnp.float32),
            pltpu.SemaphoreType.DMA((2,)),
            pltpu.SemaphoreType.DMA((2,)),
        ],
    )
    def run(t_hbm, q_hbm, o_hbm, idx_v, rows_v, gsem, osem):
        wid = jax.lax.axis_index("subcore") * 2 + jax.lax.axis_index("core")
        base = wid * BPW
        pltpu.sync_copy(t_hbm.at[pl.ds(base, BPW)], idx_v)

        @pl.loop(0, BPW, step=LANES)
        def _(c):
            sl = pl.ds(c, LANES)
            idx_v.at[sl][...] = jax.lax.rem(idx_v.at[sl][...], jnp.int32(W))

        NCH = BPW // CH

        def g_copy(i):
            b = i % 2
            return pltpu.make_async_copy(
                q_hbm.at[idx_v.at[pl.ds(i * CH, CH)]], rows_v.at[b], gsem.at[b]
            )

        def o_copy(i):
            b = i % 2
            return pltpu.make_async_copy(
                rows_v.at[b], o_hbm.at[pl.ds(base + i * CH, CH)], osem.at[b]
            )

        g_copy(0).start()
        g_copy(1).start()
        for i in range(NCH):
            g_copy(i).wait()
            o_copy(i).start()
            if i + 2 < NCH:
                o_copy(i).wait()
                g_copy(i + 2).start()
        o_copy(NCH - 2).wait()
        o_copy(NCH - 1).wait()

    return run(t32, cycleQueue)
